# baseline (speedup 1.0000x reference)
import jax
import jax.numpy as jnp
from jax.experimental import pallas as pl

CBLK = 8192


def _dense_body(ste_ref, pw_ref, uo_ref, idx_ref, coef_ref, addc_ref, out_ref):
    i = pl.program_id(0)
    lo = i * CBLK
    s = jnp.dot(ste_ref[...], pw_ref[...], preferred_element_type=jnp.float32)
    srow = s[:, 0][None, :]
    acc = uo_ref[...] + srow
    cols = lo + jax.lax.broadcasted_iota(jnp.int32, (1, CBLK), 1)
    idx = idx_ref[...]
    coef = coef_ref[...]
    addc = addc_ref[...]
    for k in range(idx.shape[1]):
        hit = idx[:, k : k + 1] == cols
        acc = jnp.where(hit, coef[:, k : k + 1] * srow + addc[:, k : k + 1], acc)
    out_ref[...] = acc


def kernel(user_embedding, station_embedding, nodes, user_id, raw_field_embed,
           user_emb_table, station_emb_table, proj_W, proj_b, theta, alpha_fields,
           fcs_W, fcs_b, fcu_W, fcu_b, mh_W1, mh_b1, mh_W2, mh_b2):
    N, D = station_emb_table.shape
    B, _, K = nodes.shape

    w = fcs_W[:, 0]
    pw = proj_W @ w
    pbw = proj_b @ w
    c0 = pbw + fcs_b[0]

    th = theta[user_id, 0]
    user_mem = (1.0 - th)[:, None] * user_embedding + th[:, None] * user_emb_table[user_id]
    u = user_mem @ fcu_W[:, 0] + fcu_b[0]
    uo = u + c0

    his = nodes[:, 0, :]
    now = nodes[:, 1, :]
    t_idx = jnp.concatenate([now, his], axis=1)
    a_t = alpha_fields[t_idx, 0]
    a_now = a_t[:, :K]
    a_his = a_t[:, K:]

    in_now = (his[:, :, None] == now[:, None, :]).any(-1)
    coef_now = 1.0 - a_now * a_now
    coef_his = jnp.where(in_now, 1.0 - a_his * a_his, 1.0 - a_his)

    w2 = mh_W2 @ w
    h = jnp.einsum("bkd,dh->bkh", raw_field_embed[his], mh_W1) + mh_b1
    h = jax.nn.leaky_relu(h, negative_slope=0.01)
    mlp_d = h @ w2 + mh_b2 @ w

    coef = jnp.concatenate([coef_now, coef_his], axis=1)
    add = jnp.concatenate([jnp.zeros_like(coef_now), a_his * mlp_d], axis=1)
    addc = coef * pbw + add + fcs_b[0] + u[:, None]

    nb = pl.cdiv(N, CBLK)
    out = pl.pallas_call(
        _dense_body,
        grid=(nb,),
        in_specs=[
            pl.BlockSpec((CBLK, D), lambda i: (i, 0)),
            pl.BlockSpec((D, 1), lambda i: (0, 0)),
            pl.BlockSpec((B, 1), lambda i: (0, 0)),
            pl.BlockSpec((B, 2 * K), lambda i: (0, 0)),
            pl.BlockSpec((B, 2 * K), lambda i: (0, 0)),
            pl.BlockSpec((B, 2 * K), lambda i: (0, 0)),
        ],
        out_specs=pl.BlockSpec((B, CBLK), lambda i: (0, i)),
        out_shape=jax.ShapeDtypeStruct((B, N), jnp.float32),
    )(station_emb_table, pw[:, None], uo[:, None], t_idx, coef, addc)
    return out


# speedup vs baseline: 1.2201x; 1.2201x over previous
import jax
import jax.numpy as jnp
from jax.experimental import pallas as pl

CBLK = 8192


def _dense_body(ste_ref, pw_ref, uo_ref, idx_ref, coef_ref, addc_ref, out_ref):
    i = pl.program_id(0)
    lo = i * CBLK
    s = jnp.dot(ste_ref[...], pw_ref[...], preferred_element_type=jnp.float32)
    srow = s[:, 0][None, :]
    cols = lo + jax.lax.broadcasted_iota(jnp.int32, (1, CBLK), 1)
    idx = idx_ref[...]
    coef = coef_ref[...]
    addc = addc_ref[...]
    B = idx.shape[0]
    cc = jnp.ones((B, CBLK), jnp.float32)
    aa = jnp.broadcast_to(uo_ref[...], (B, CBLK))
    for k in range(idx.shape[1]):
        hit = idx[:, k : k + 1] == cols
        cc = jnp.where(hit, coef[:, k : k + 1], cc)
        aa = jnp.where(hit, addc[:, k : k + 1], aa)
    out_ref[...] = cc * srow + aa


def kernel(user_embedding, station_embedding, nodes, user_id, raw_field_embed,
           user_emb_table, station_emb_table, proj_W, proj_b, theta, alpha_fields,
           fcs_W, fcs_b, fcu_W, fcu_b, mh_W1, mh_b1, mh_W2, mh_b2):
    N, D = station_emb_table.shape
    B, _, K = nodes.shape

    w = fcs_W[:, 0]
    pw = proj_W @ w
    pbw = proj_b @ w
    c0 = pbw + fcs_b[0]

    th = theta[user_id, 0]
    user_mem = (1.0 - th)[:, None] * user_embedding + th[:, None] * user_emb_table[user_id]
    u = user_mem @ fcu_W[:, 0] + fcu_b[0]
    uo = u + c0

    his = nodes[:, 0, :]
    now = nodes[:, 1, :]
    t_idx = jnp.concatenate([now, his], axis=1)
    a_t = alpha_fields[t_idx, 0]
    a_now = a_t[:, :K]
    a_his = a_t[:, K:]

    in_now = (his[:, :, None] == now[:, None, :]).any(-1)
    coef_now = 1.0 - a_now * a_now
    coef_his = jnp.where(in_now, 1.0 - a_his * a_his, 1.0 - a_his)

    w2 = mh_W2 @ w
    h = jnp.einsum("bkd,dh->bkh", raw_field_embed[his], mh_W1) + mh_b1
    h = jax.nn.leaky_relu(h, negative_slope=0.01)
    mlp_d = h @ w2 + mh_b2 @ w

    coef = jnp.concatenate([coef_now, coef_his], axis=1)
    add = jnp.concatenate([jnp.zeros_like(coef_now), a_his * mlp_d], axis=1)
    addc = coef * pbw + add + fcs_b[0] + u[:, None]

    nb = pl.cdiv(N, CBLK)
    out = pl.pallas_call(
        _dense_body,
        grid=(nb,),
        in_specs=[
            pl.BlockSpec((CBLK, D), lambda i: (i, 0)),
            pl.BlockSpec((D, 1), lambda i: (0, 0)),
            pl.BlockSpec((B, 1), lambda i: (0, 0)),
            pl.BlockSpec((B, 2 * K), lambda i: (0, 0)),
            pl.BlockSpec((B, 2 * K), lambda i: (0, 0)),
            pl.BlockSpec((B, 2 * K), lambda i: (0, 0)),
        ],
        out_specs=pl.BlockSpec((B, CBLK), lambda i: (0, i)),
        out_shape=jax.ShapeDtypeStruct((B, N), jnp.float32),
    )(station_emb_table, pw[:, None], uo[:, None], t_idx, coef, addc)
    return out
